# baseline (speedup 1.0000x reference)
import jax
import jax.numpy as jnp
from jax import lax
from jax.experimental import pallas as pl
from jax.experimental.pallas import tpu as pltpu
from jax.experimental.pallas import tpu_sc as plsc

_N = 10000
_E = 320000
_D = 128
_G = 64
_EPS = 1e-5

_NSC = 2
_NTILE = 16
_NW = _NSC * _NTILE
_N_PAD = 10112
_RPT = _N_PAD // _NTILE
_DUMMY = 10008
_CHUNK = 128
_EPW = _E // _NW
_CPT = 80
_EPT = _CPT * _CHUNK
_GRP = 8
_NGRP = _CPT // _GRP
_DEG_STEPS = _EPW // 16


def _sc_deg_body(dst_hbm, zeros_hbm, out_hbm, dbuf, acc):
    c = lax.axis_index("c")
    s = lax.axis_index("s")
    w = c * _NTILE + s
    pltpu.sync_copy(dst_hbm.at[pl.ds(w * _EPW, _EPW)], dbuf)
    pltpu.sync_copy(zeros_hbm, acc)
    ones = jnp.ones((16,), jnp.float32)

    def step(i, carry):
        idx = dbuf[pl.ds(i * 16, 16)]
        plsc.addupdate_scatter(acc, [idx], ones)
        return carry

    lax.fori_loop(0, _DEG_STEPS, step, 0)
    pltpu.sync_copy(acc, out_hbm.at[w])


def _sc_agg_body(y_hbm, src_hbm, dst_hbm, zeros_hbm, out_hbm,
                 src_v, ring, buf_a, buf_b, acc_sh, sem_a, sem_b, rsem_a,
                 rsem_b):
    c = lax.axis_index("c")
    s = lax.axis_index("s")
    pltpu.sync_copy(zeros_hbm, acc_sh.at[pl.ds(s * _RPT, _RPT)])
    pltpu.sync_copy(src_hbm.at[c, s], src_v)
    plsc.subcore_barrier()

    pltpu.async_copy(dst_hbm.at[c, s, 0], ring.at[pl.ds(0, _GRP)], rsem_a)
    pltpu.async_copy(dst_hbm.at[c, s, 1], ring.at[pl.ds(_GRP, _GRP)], rsem_b)
    pltpu.async_copy(y_hbm.at[src_v.at[0]], buf_a, sem_a)
    pltpu.async_copy(y_hbm.at[src_v.at[1]], buf_b, sem_b)

    def half(i, carry):
        for h in range(2):
            m = 2 * i + h
            base = m * _GRP
            ring_slot = ring.at[pl.ds(h * _GRP, _GRP)]
            rsem = rsem_a if h == 0 else rsem_b
            pltpu.make_async_copy(dst_hbm.at[c, s, m], ring_slot,
                                  rsem).wait()
            for k in range(_GRP):
                j = base + k
                nxt = lax.min(j + 2, _CPT - 1)
                buf, sem = (buf_a, sem_a) if k % 2 == 0 else (buf_b, sem_b)
                pltpu.make_async_copy(y_hbm.at[src_v.at[j]], buf,
                                      sem).wait()
                pltpu.sync_copy(buf, acc_sh.at[ring_slot.at[k]], add=True)
                pltpu.async_copy(y_hbm.at[src_v.at[nxt]], buf, sem)
            nxt_m = lax.min(m + 2, _NGRP - 1)
            pltpu.async_copy(dst_hbm.at[c, s, nxt_m], ring_slot, rsem)
        return carry

    lax.fori_loop(0, _NGRP // 2, half, 0)
    pltpu.make_async_copy(y_hbm.at[src_v.at[0]], buf_a, sem_a).wait()
    pltpu.make_async_copy(y_hbm.at[src_v.at[0]], buf_b, sem_b).wait()
    pltpu.make_async_copy(dst_hbm.at[c, s, 0], ring.at[pl.ds(0, _GRP)],
                          rsem_a).wait()
    pltpu.make_async_copy(dst_hbm.at[c, s, 0], ring.at[pl.ds(_GRP, _GRP)],
                          rsem_b).wait()
    plsc.subcore_barrier()
    pltpu.sync_copy(acc_sh.at[pl.ds(s * _RPT, _RPT)],
                    out_hbm.at[c, pl.ds(s * _RPT, _RPT)])


def _get_mesh():
    return plsc.VectorSubcoreMesh(core_axis_name="c", subcore_axis_name="s")


def _sc_deg(dst, zeros_n):
    fn = pl.kernel(
        _sc_deg_body,
        out_type=jax.ShapeDtypeStruct((_NW, _N_PAD), jnp.float32),
        mesh=_get_mesh(),
        scratch_types=[
            pltpu.VMEM((_EPW,), jnp.int32),
            pltpu.VMEM((_N_PAD,), jnp.float32),
        ],
        compiler_params=pltpu.CompilerParams(needs_layout_passes=False),
    )
    return fn(dst, zeros_n)


def _sc_agg(y, src_pp, dst_pp, zeros_c):
    fn = pl.kernel(
        _sc_agg_body,
        out_type=jax.ShapeDtypeStruct((_NSC, _N_PAD, _D), jnp.float32),
        mesh=_get_mesh(),
        scratch_types=[
            pltpu.VMEM((_CPT, _CHUNK), jnp.int32),
            pltpu.VMEM((2 * _GRP, _CHUNK), jnp.int32),
            pltpu.VMEM((_CHUNK, _D), jnp.float32),
            pltpu.VMEM((_CHUNK, _D), jnp.float32),
            pltpu.VMEM_SHARED((_N_PAD, _D), jnp.float32),
            pltpu.SemaphoreType.DMA,
            pltpu.SemaphoreType.DMA,
            pltpu.SemaphoreType.DMA,
            pltpu.SemaphoreType.DMA,
        ],
    )
    return fn(y, src_pp, dst_pp, zeros_c)


def _mm(a, b):
    return jnp.dot(a.astype(jnp.bfloat16), b.astype(jnp.bfloat16),
                   preferred_element_type=jnp.float32)


def _bn_relu(t, g, be):
    mu = jnp.mean(t, axis=0)
    var = jnp.mean((t - mu) ** 2, axis=0)
    return jnp.maximum((t - mu) * lax.rsqrt(var + _EPS) * g + be, 0.0)


def _tc_pre_body(x_ref, w1_ref, wproj_ref, degp_ref,
                 dinv_ref, id_ref, y_ref):
    deg = jnp.sum(degp_ref[...], axis=0)[:_N] + 1.0
    dinv = lax.rsqrt(deg)
    dinv_ref[...] = dinv
    x = x_ref[...]
    id_ref[...] = _mm(x, wproj_ref[...])
    xw = _mm(x, w1_ref[...])
    y_ref[pl.ds(0, _N), :] = xw * dinv[:, None]


def _tc_mid_body(part_ref, y_ref, id_ref, dinv_ref, b_ref, g_ref, be_ref,
                 wn_ref, h_ref, ynext_ref):
    dinv = dinv_ref[...]
    s = part_ref[0, :_N, :] + part_ref[1, :_N, :]
    t = (s + y_ref[pl.ds(0, _N), :]) * dinv[:, None] + b_ref[...]
    h = _bn_relu(t, g_ref[...], be_ref[...]) + id_ref[...]
    h_ref[...] = h
    hw = _mm(h, wn_ref[...])
    ynext_ref[pl.ds(0, _N), :] = hw * dinv[:, None]


def _tc_post_body(part_ref, y_ref, id_ref, dinv_ref, b_ref, g_ref, be_ref,
                  batch_ref, wpre_ref, bpre_ref, gf_ref, bef_ref,
                  wout_ref, bout_ref, out_ref):
    dinv = dinv_ref[...]
    s = part_ref[0, :_N, :] + part_ref[1, :_N, :]
    t = (s + y_ref[pl.ds(0, _N), :]) * dinv[:, None] + b_ref[...]
    h = _bn_relu(t, g_ref[...], be_ref[...]) + id_ref[...]
    gids = lax.broadcasted_iota(jnp.int32, (_G, _N), 0)
    oh = (batch_ref[...][None, :] == gids).astype(jnp.float32)
    sums = jnp.dot(oh, h, preferred_element_type=jnp.float32,
                   precision=lax.Precision.HIGHEST)
    cnt = jnp.sum(oh, axis=1)
    pooled = sums / jnp.maximum(cnt, 1.0)[:, None]
    h2 = _mm(pooled, wpre_ref[...]) + bpre_ref[...]
    h2 = _bn_relu(h2, gf_ref[...], bef_ref[...])
    out_ref[...] = _mm(h2, wout_ref[...]) + bout_ref[...]


def _tc_pre(x, w1, wproj, degp):
    return pl.pallas_call(
        _tc_pre_body,
        out_shape=[
            jax.ShapeDtypeStruct((_N,), jnp.float32),
            jax.ShapeDtypeStruct((_N, _D), jnp.float32),
            jax.ShapeDtypeStruct((_N_PAD, _D), jnp.float32),
        ],
    )(x, w1, wproj, degp)


def _tc_mid(part, y, ident, dinv, b, g, be, wn):
    return pl.pallas_call(
        _tc_mid_body,
        out_shape=[
            jax.ShapeDtypeStruct((_N, _D), jnp.float32),
            jax.ShapeDtypeStruct((_N_PAD, _D), jnp.float32),
        ],
    )(part, y, ident, dinv, b, g, be, wn)


def _tc_post(part, y, ident, dinv, b, g, be, batch,
             wpre, bpre, gf, bef, wout, bout):
    return pl.pallas_call(
        _tc_post_body,
        out_shape=jax.ShapeDtypeStruct((_G, 1), jnp.float32),
    )(part, y, ident, dinv, b, g, be, batch,
      wpre, bpre, gf, bef, wout, bout)


def kernel(x, edge_index, batch, W1, b1, W2, b2, W3, b3, W4, b4, Wproj,
           g1, be1, g2, be2, g3, be3, g4, be4, Wpre, bpre, gf, bef,
           Wout, bout):
    src = edge_index[0]
    dst = edge_index[1]
    srcp = jnp.pad(src.reshape(_NW, _EPW), ((0, 0), (0, _EPT - _EPW)),
                   constant_values=_DUMMY)
    dstp = jnp.pad(dst.reshape(_NW, _EPW), ((0, 0), (0, _EPT - _EPW)),
                   constant_values=_DUMMY)
    src_pp = srcp.reshape(_NSC, _NTILE, _CPT, _CHUNK)
    dst_pp = dstp.reshape(_NSC, _NTILE, _NGRP, _GRP, _CHUNK)
    zeros_n = jnp.zeros((_N_PAD,), jnp.float32)
    zeros_c = jnp.zeros((_RPT, _D), jnp.float32)

    degp = _sc_deg(dst, zeros_n)
    dinv, ident, y = _tc_pre(x, W1, Wproj, degp)

    part = _sc_agg(y, src_pp, dst_pp, zeros_c)
    ident, y = _tc_mid(part, y, ident, dinv, b1, g1, be1, W2)

    part = _sc_agg(y, src_pp, dst_pp, zeros_c)
    ident, y = _tc_mid(part, y, ident, dinv, b2, g2, be2, W3)

    part = _sc_agg(y, src_pp, dst_pp, zeros_c)
    ident, y = _tc_mid(part, y, ident, dinv, b3, g3, be3, W4)

    part = _sc_agg(y, src_pp, dst_pp, zeros_c)
    return _tc_post(part, y, ident, dinv, b4, g4, be4, batch,
                    Wpre, bpre, gf, bef, Wout, bout)


# speedup vs baseline: 1.5025x; 1.5025x over previous
import jax
import jax.numpy as jnp
from jax import lax
from jax.experimental import pallas as pl
from jax.experimental.pallas import tpu as pltpu
from jax.experimental.pallas import tpu_sc as plsc

_N = 10000
_E = 320000
_D = 128
_G = 64
_EPS = 1e-5

_NSC = 2
_NTILE = 16
_NW = _NSC * _NTILE
_N_PAD = 10112
_RPT = _N_PAD // _NTILE
_DUMMY = 10008
_CHUNK = 128
_EPW = _E // _NW
_CPT = 80
_EPT = _CPT * _CHUNK
_GRP = 8
_NGRP = _CPT // _GRP
_CBYTES = _CHUNK * _D * 4
_RBYTES = _GRP * _CHUNK * 4
_DEG_STEPS = _EPW // 16


def _sc_deg_body(dst_hbm, zeros_hbm, out_hbm, dbuf, acc):
    c = lax.axis_index("c")
    s = lax.axis_index("s")
    w = c * _NTILE + s
    pltpu.sync_copy(dst_hbm.at[pl.ds(w * _EPW, _EPW)], dbuf)
    pltpu.sync_copy(zeros_hbm, acc)
    ones = jnp.ones((16,), jnp.float32)

    def step(i, carry):
        idx = dbuf[pl.ds(i * 16, 16)]
        plsc.addupdate_scatter(acc, [idx], ones)
        return carry

    lax.fori_loop(0, _DEG_STEPS, step, 0)
    pltpu.sync_copy(acc, out_hbm.at[w])


def _sc_agg_body(y_hbm, src_hbm, dst_hbm, zeros_hbm, out_hbm,
                 src_v, dst_v, buf, acc_sh):
    c = lax.axis_index("c")
    s = lax.axis_index("s")
    pltpu.sync_copy(zeros_hbm, acc_sh.at[pl.ds(s * _RPT, _RPT)])
    pltpu.sync_copy(src_hbm.at[c, s], src_v)
    pltpu.sync_copy(dst_hbm.at[c, s], dst_v)
    plsc.subcore_barrier()

    def chunk(j, carry):
        pltpu.sync_copy(y_hbm.at[src_v.at[j]], buf)
        pltpu.sync_copy(buf, acc_sh.at[dst_v.at[j]], add=True)
        return carry

    lax.fori_loop(0, _CPT, chunk, 0)
    plsc.subcore_barrier()
    pltpu.sync_copy(acc_sh.at[pl.ds(s * _RPT, _RPT)],
                    out_hbm.at[c, pl.ds(s * _RPT, _RPT)])


def _get_mesh():
    return plsc.VectorSubcoreMesh(core_axis_name="c", subcore_axis_name="s")


def _sc_deg(dst, zeros_n):
    fn = pl.kernel(
        _sc_deg_body,
        out_type=jax.ShapeDtypeStruct((_NW, _N_PAD), jnp.float32),
        mesh=_get_mesh(),
        scratch_types=[
            pltpu.VMEM((_EPW,), jnp.int32),
            pltpu.VMEM((_N_PAD,), jnp.float32),
        ],
        compiler_params=pltpu.CompilerParams(needs_layout_passes=False),
    )
    return fn(dst, zeros_n)


def _sc_agg(y, src_pp, dst_pp, zeros_c):
    fn = pl.kernel(
        _sc_agg_body,
        out_type=jax.ShapeDtypeStruct((_NSC, _N_PAD, _D), jnp.float32),
        mesh=_get_mesh(),
        scratch_types=[
            pltpu.VMEM((_CPT, _CHUNK), jnp.int32),
            pltpu.VMEM((_CPT, _CHUNK), jnp.int32),
            pltpu.VMEM((_CHUNK, _D), jnp.float32),
            pltpu.VMEM_SHARED((_N_PAD, _D), jnp.float32),
        ],
    )
    return fn(y, src_pp, dst_pp, zeros_c)


def _mm(a, b):
    return jnp.dot(a.astype(jnp.bfloat16), b.astype(jnp.bfloat16),
                   preferred_element_type=jnp.float32)


def _bn_relu(t, g, be):
    mu = jnp.mean(t, axis=0)
    var = jnp.mean((t - mu) ** 2, axis=0)
    return jnp.maximum((t - mu) * lax.rsqrt(var + _EPS) * g + be, 0.0)


def _tc_pre_body(x_ref, w1_ref, wproj_ref, degp_ref,
                 dinv_ref, id_ref, y_ref):
    deg = jnp.sum(degp_ref[...], axis=0)[:_N] + 1.0
    dinv = lax.rsqrt(deg)
    dinv_ref[...] = dinv
    x = x_ref[...]
    id_ref[...] = _mm(x, wproj_ref[...])
    xw = _mm(x, w1_ref[...])
    y_ref[pl.ds(0, _N), :] = xw * dinv[:, None]


def _tc_mid_body(part_ref, y_ref, id_ref, dinv_ref, b_ref, g_ref, be_ref,
                 wn_ref, h_ref, ynext_ref):
    dinv = dinv_ref[...]
    s = part_ref[0, :_N, :] + part_ref[1, :_N, :]
    t = (s + y_ref[pl.ds(0, _N), :]) * dinv[:, None] + b_ref[...]
    h = _bn_relu(t, g_ref[...], be_ref[...]) + id_ref[...]
    h_ref[...] = h
    hw = _mm(h, wn_ref[...])
    ynext_ref[pl.ds(0, _N), :] = hw * dinv[:, None]


def _tc_post_body(part_ref, y_ref, id_ref, dinv_ref, b_ref, g_ref, be_ref,
                  batch_ref, wpre_ref, bpre_ref, gf_ref, bef_ref,
                  wout_ref, bout_ref, out_ref):
    dinv = dinv_ref[...]
    s = part_ref[0, :_N, :] + part_ref[1, :_N, :]
    t = (s + y_ref[pl.ds(0, _N), :]) * dinv[:, None] + b_ref[...]
    h = _bn_relu(t, g_ref[...], be_ref[...]) + id_ref[...]
    gids = lax.broadcasted_iota(jnp.int32, (_G, _N), 0)
    oh = (batch_ref[...][None, :] == gids).astype(jnp.float32)
    sums = jnp.dot(oh, h, preferred_element_type=jnp.float32,
                   precision=lax.Precision.HIGHEST)
    cnt = jnp.sum(oh, axis=1)
    pooled = sums / jnp.maximum(cnt, 1.0)[:, None]
    h2 = _mm(pooled, wpre_ref[...]) + bpre_ref[...]
    h2 = _bn_relu(h2, gf_ref[...], bef_ref[...])
    out_ref[...] = _mm(h2, wout_ref[...]) + bout_ref[...]


def _tc_pre(x, w1, wproj, degp):
    return pl.pallas_call(
        _tc_pre_body,
        out_shape=[
            jax.ShapeDtypeStruct((_N,), jnp.float32),
            jax.ShapeDtypeStruct((_N, _D), jnp.float32),
            jax.ShapeDtypeStruct((_N_PAD, _D), jnp.float32),
        ],
    )(x, w1, wproj, degp)


def _tc_mid(part, y, ident, dinv, b, g, be, wn):
    return pl.pallas_call(
        _tc_mid_body,
        out_shape=[
            jax.ShapeDtypeStruct((_N, _D), jnp.float32),
            jax.ShapeDtypeStruct((_N_PAD, _D), jnp.float32),
        ],
    )(part, y, ident, dinv, b, g, be, wn)


def _tc_post(part, y, ident, dinv, b, g, be, batch,
             wpre, bpre, gf, bef, wout, bout):
    return pl.pallas_call(
        _tc_post_body,
        out_shape=jax.ShapeDtypeStruct((_G, 1), jnp.float32),
    )(part, y, ident, dinv, b, g, be, batch,
      wpre, bpre, gf, bef, wout, bout)


def kernel(x, edge_index, batch, W1, b1, W2, b2, W3, b3, W4, b4, Wproj,
           g1, be1, g2, be2, g3, be3, g4, be4, Wpre, bpre, gf, bef,
           Wout, bout):
    src = edge_index[0]
    dst = edge_index[1]
    srcp = jnp.pad(src.reshape(_NW, _EPW), ((0, 0), (0, _EPT - _EPW)),
                   constant_values=_DUMMY)
    dstp = jnp.pad(dst.reshape(_NW, _EPW), ((0, 0), (0, _EPT - _EPW)),
                   constant_values=_DUMMY)
    src_pp = srcp.reshape(_NSC, _NTILE, _CPT, _CHUNK)
    dst_pp = dstp.reshape(_NSC, _NTILE, _CPT, _CHUNK)
    zeros_n = jnp.zeros((_N_PAD,), jnp.float32)
    zeros_c = jnp.zeros((_RPT, _D), jnp.float32)

    degp = _sc_deg(dst, zeros_n)
    dinv, ident, y = _tc_pre(x, W1, Wproj, degp)

    part = _sc_agg(y, src_pp, dst_pp, zeros_c)
    ident, y = _tc_mid(part, y, ident, dinv, b1, g1, be1, W2)

    part = _sc_agg(y, src_pp, dst_pp, zeros_c)
    ident, y = _tc_mid(part, y, ident, dinv, b2, g2, be2, W3)

    part = _sc_agg(y, src_pp, dst_pp, zeros_c)
    ident, y = _tc_mid(part, y, ident, dinv, b3, g3, be3, W4)

    part = _sc_agg(y, src_pp, dst_pp, zeros_c)
    return _tc_post(part, y, ident, dinv, b4, g4, be4, batch,
                    Wpre, bpre, gf, bef, Wout, bout)


# speedup vs baseline: 2.2065x; 1.4685x over previous
import jax
import jax.numpy as jnp
from jax import lax
from jax.experimental import pallas as pl
from jax.experimental.pallas import tpu as pltpu
from jax.experimental.pallas import tpu_sc as plsc

_N = 10000
_E = 320000
_D = 128
_G = 64
_EPS = 1e-5

_NSC = 2
_NTILE = 16
_NW = _NSC * _NTILE
_N_PAD = 10240
_RPT = _N_PAD // _NTILE
_DUMMY = 10008
_CHUNK = 128
_EPW = _E // _NW
_CPT = 79
_EPT = _CPT * _CHUNK
_GRP = 8
_NGRP = _CPT // _GRP
_CBYTES = _CHUNK * _D * 4
_RBYTES = _GRP * _CHUNK * 4
_DEG_STEPS = _EPW // 16


def _sc_deg_body(dst_hbm, zeros_hbm, out_hbm, dbuf, acc):
    c = lax.axis_index("c")
    s = lax.axis_index("s")
    w = c * _NTILE + s
    pltpu.sync_copy(dst_hbm.at[pl.ds(w * _EPW, _EPW)], dbuf)
    pltpu.sync_copy(zeros_hbm, acc)
    ones = jnp.ones((16,), jnp.float32)

    def step(i, carry):
        idx = dbuf[pl.ds(i * 16, 16)]
        plsc.addupdate_scatter(acc, [idx], ones)
        return carry

    lax.fori_loop(0, _DEG_STEPS, step, 0)
    pltpu.sync_copy(acc, out_hbm.at[w])


def _sc_agg_body(y_hbm, src_hbm, dst_hbm, zeros_hbm, out_hbm,
                 src_v, dst_v, buf, acc_sh):
    c = lax.axis_index("c")
    s = lax.axis_index("s")
    pltpu.sync_copy(zeros_hbm, acc_sh.at[pl.ds(s * _RPT, _RPT)])
    pltpu.sync_copy(src_hbm.at[c, s], src_v)
    pltpu.sync_copy(dst_hbm.at[c, s], dst_v)
    plsc.subcore_barrier()

    def chunk(j, carry):
        pltpu.sync_copy(y_hbm.at[src_v.at[j]], buf)
        pltpu.sync_copy(buf, acc_sh.at[dst_v.at[j]], add=True)
        return carry

    lax.fori_loop(0, _CPT, chunk, 0)
    plsc.subcore_barrier()
    pltpu.sync_copy(acc_sh.at[pl.ds(s * _RPT, _RPT)],
                    out_hbm.at[c, pl.ds(s * _RPT, _RPT)])


def _get_mesh():
    return plsc.VectorSubcoreMesh(core_axis_name="c", subcore_axis_name="s")


def _sc_deg(dst, zeros_n):
    fn = pl.kernel(
        _sc_deg_body,
        out_type=jax.ShapeDtypeStruct((_NW, _N_PAD), jnp.float32),
        mesh=_get_mesh(),
        scratch_types=[
            pltpu.VMEM((_EPW,), jnp.int32),
            pltpu.VMEM((_N_PAD,), jnp.float32),
        ],
        compiler_params=pltpu.CompilerParams(needs_layout_passes=False),
    )
    return fn(dst, zeros_n)


def _sc_agg(y, src_pp, dst_pp, zeros_c):
    fn = pl.kernel(
        _sc_agg_body,
        out_type=jax.ShapeDtypeStruct((_NSC, _N_PAD, _D), jnp.float32),
        mesh=_get_mesh(),
        scratch_types=[
            pltpu.VMEM((_CPT, _CHUNK), jnp.int32),
            pltpu.VMEM((_CPT, _CHUNK), jnp.int32),
            pltpu.VMEM((_CHUNK, _D), jnp.float32),
            pltpu.VMEM_SHARED((_N_PAD, _D), jnp.float32),
        ],
    )
    return fn(y, src_pp, dst_pp, zeros_c)


def _mm(a, b):
    return jnp.dot(a.astype(jnp.bfloat16), b.astype(jnp.bfloat16),
                   preferred_element_type=jnp.float32)


def _bn_relu(t, g, be):
    mu = jnp.mean(t, axis=0)
    var = jnp.mean((t - mu) ** 2, axis=0)
    return jnp.maximum((t - mu) * lax.rsqrt(var + _EPS) * g + be, 0.0)


def _tc_pre_body(x_ref, w1_ref, wproj_ref, degp_ref,
                 dinv_ref, id_ref, y_ref):
    deg = jnp.sum(degp_ref[...], axis=0)[:_N] + 1.0
    dinv = lax.rsqrt(deg)
    dinv_ref[...] = dinv
    x = x_ref[...]
    id_ref[...] = _mm(x, wproj_ref[...])
    xw = _mm(x, w1_ref[...])
    y_ref[pl.ds(0, _N), :] = xw * dinv[:, None]


def _tc_mid_body(part_ref, y_ref, id_ref, dinv_ref, b_ref, g_ref, be_ref,
                 wn_ref, h_ref, ynext_ref):
    dinv = dinv_ref[...]
    s = part_ref[0, :_N, :] + part_ref[1, :_N, :]
    t = (s + y_ref[pl.ds(0, _N), :]) * dinv[:, None] + b_ref[...]
    h = _bn_relu(t, g_ref[...], be_ref[...]) + id_ref[...]
    h_ref[...] = h
    hw = _mm(h, wn_ref[...])
    ynext_ref[pl.ds(0, _N), :] = hw * dinv[:, None]


def _tc_post_body(part_ref, y_ref, id_ref, dinv_ref, b_ref, g_ref, be_ref,
                  batch_ref, wpre_ref, bpre_ref, gf_ref, bef_ref,
                  wout_ref, bout_ref, out_ref):
    dinv = dinv_ref[...]
    s = part_ref[0, :_N, :] + part_ref[1, :_N, :]
    t = (s + y_ref[pl.ds(0, _N), :]) * dinv[:, None] + b_ref[...]
    h = _bn_relu(t, g_ref[...], be_ref[...]) + id_ref[...]
    gids = lax.broadcasted_iota(jnp.int32, (_G, _N), 0)
    oh = (batch_ref[...][None, :] == gids).astype(jnp.float32)
    sums = jnp.dot(oh, h, preferred_element_type=jnp.float32,
                   precision=lax.Precision.HIGHEST)
    cnt = jnp.sum(oh, axis=1)
    pooled = sums / jnp.maximum(cnt, 1.0)[:, None]
    h2 = _mm(pooled, wpre_ref[...]) + bpre_ref[...]
    h2 = _bn_relu(h2, gf_ref[...], bef_ref[...])
    out_ref[...] = _mm(h2, wout_ref[...]) + bout_ref[...]


def _tc_pre(x, w1, wproj, degp):
    return pl.pallas_call(
        _tc_pre_body,
        out_shape=[
            jax.ShapeDtypeStruct((_N,), jnp.float32),
            jax.ShapeDtypeStruct((_N, _D), jnp.float32),
            jax.ShapeDtypeStruct((_N_PAD, _D), jnp.float32),
        ],
    )(x, w1, wproj, degp)


def _tc_mid(part, y, ident, dinv, b, g, be, wn):
    return pl.pallas_call(
        _tc_mid_body,
        out_shape=[
            jax.ShapeDtypeStruct((_N, _D), jnp.float32),
            jax.ShapeDtypeStruct((_N_PAD, _D), jnp.float32),
        ],
    )(part, y, ident, dinv, b, g, be, wn)


def _tc_post(part, y, ident, dinv, b, g, be, batch,
             wpre, bpre, gf, bef, wout, bout):
    return pl.pallas_call(
        _tc_post_body,
        out_shape=jax.ShapeDtypeStruct((_G, 1), jnp.float32),
    )(part, y, ident, dinv, b, g, be, batch,
      wpre, bpre, gf, bef, wout, bout)


def kernel(x, edge_index, batch, W1, b1, W2, b2, W3, b3, W4, b4, Wproj,
           g1, be1, g2, be2, g3, be3, g4, be4, Wpre, bpre, gf, bef,
           Wout, bout):
    src = edge_index[0]
    dst = edge_index[1]
    srcp = jnp.pad(src.reshape(_NW, _EPW), ((0, 0), (0, _EPT - _EPW)),
                   constant_values=_DUMMY)
    dstp = jnp.pad(dst.reshape(_NW, _EPW), ((0, 0), (0, _EPT - _EPW)),
                   constant_values=_DUMMY)
    src_pp = srcp.reshape(_NSC, _NTILE, _CPT, _CHUNK)
    dst_pp = dstp.reshape(_NSC, _NTILE, _CPT, _CHUNK)
    zeros_n = jnp.zeros((_N_PAD,), jnp.float32)
    zeros_c = jnp.zeros((_RPT, _D), jnp.float32)

    degp = _sc_deg(dst, zeros_n)
    dinv, ident, y = _tc_pre(x, W1, Wproj, degp)

    part = _sc_agg(y, src_pp, dst_pp, zeros_c)
    ident, y = _tc_mid(part, y, ident, dinv, b1, g1, be1, W2)

    part = _sc_agg(y, src_pp, dst_pp, zeros_c)
    ident, y = _tc_mid(part, y, ident, dinv, b2, g2, be2, W3)

    part = _sc_agg(y, src_pp, dst_pp, zeros_c)
    ident, y = _tc_mid(part, y, ident, dinv, b3, g3, be3, W4)

    part = _sc_agg(y, src_pp, dst_pp, zeros_c)
    return _tc_post(part, y, ident, dinv, b4, g4, be4, batch,
                    Wpre, bpre, gf, bef, Wout, bout)
